# Initial kernel scaffold; baseline (speedup 1.0000x reference)
import jax
import jax.numpy as jnp
from jax.experimental import pallas as pl


def kernel(x, edge_index, edge_attr, node_ids, Wm0, bm0, Wp0, bp0, Wm1, bm1, Wp1, bp1):
    raise NotImplementedError("write your pallas kernel here")


# speedup vs baseline: 1.5720x; 1.5720x over previous
import functools

import jax
import jax.numpy as jnp
from jax import lax
from jax.experimental import pallas as pl
from jax.experimental.pallas import tpu as pltpu
from jax.experimental.pallas import tpu_sc as plsc

N = 10000
E = 320000
D = 128
ED = 16
ND = 20

NC = 2
NS = 16
NW = NC * NS

NPAD = 10240
RW = NPAD // NW
RB = RW // 2
NBUCKET = 2 * NW

CH = 2000
VPC = CH // 16
NCHUNK = E // CH
STG = 2032
FLUSH = 2016
ROWCAP = E + 4096
EPAD = 321536
SENT = E
CB = 128
BIGDST = 1 << 20


def _s0_body(dst_ref, eids_ref, counts_ref, dbuf, stg0, stg1, cbuf):
    c = lax.axis_index("c")
    s = lax.axis_index("s")
    wid = s * NC + c
    b0 = wid * 2
    b1 = wid * 2 + 1
    lo0 = wid * RW
    hi0 = lo0 + RB
    hi1 = lo0 + RW

    zero16 = jnp.zeros((16,), jnp.int32)

    def zi(i, carry):
        stg0[pl.ds(i * 16, 16)] = zero16
        stg1[pl.ds(i * 16, 16)] = zero16
        return carry

    lax.fori_loop(0, STG // 16, zi, 0)

    iota = lax.iota(jnp.int32, 16)
    sent16 = jnp.full((16,), SENT, jnp.int32)

    def chunk_body(ci, carry):
        k0, k1 = carry
        pltpu.sync_copy(dst_ref.at[pl.ds(ci * CH, CH)], dbuf)

        def vec_body(i, kk):
            kc0, kc1 = kk
            d = dbuf[pl.ds(i * 16, 16)]
            eid = ci * CH + i * 16 + iota
            m0 = (d >= lo0) & (d < hi0)
            m1 = (d >= hi0) & (d < hi1)
            cs0 = plsc.cumsum(m0.astype(jnp.int32))
            cs1 = plsc.cumsum(m1.astype(jnp.int32))
            plsc.store_scatter(stg0, [kc0 + cs0 - 1], eid, mask=m0)
            plsc.store_scatter(stg1, [kc1 + cs1 - 1], eid, mask=m1)
            return kc0 + cs0[15], kc1 + cs1[15]

        kc0, kc1 = lax.fori_loop(0, VPC, vec_body,
                                 (jnp.int32(0), jnp.int32(0)))
        stg0[pl.ds(kc0, 16)] = sent16
        stg1[pl.ds(kc1, 16)] = sent16
        pltpu.sync_copy(stg0.at[pl.ds(0, FLUSH)],
                        eids_ref.at[pl.ds(pl.multiple_of(b0 * ROWCAP + k0, 8), FLUSH)])
        pltpu.sync_copy(stg1.at[pl.ds(0, FLUSH)],
                        eids_ref.at[pl.ds(pl.multiple_of(b1 * ROWCAP + k1, 8), FLUSH)])
        k0 = k0 + ((kc0 + 7) & (-8))
        k1 = k1 + ((kc1 + 7) & (-8))
        return k0, k1

    k0, k1 = lax.fori_loop(0, NCHUNK, chunk_body,
                           (jnp.int32(0), jnp.int32(0)))

    def sfill(i, carry):
        stg0[pl.ds(i * 16, 16)] = sent16
        return carry

    lax.fori_loop(0, STG // 16, sfill, 0)
    pltpu.sync_copy(stg0.at[pl.ds(0, FLUSH)],
                    eids_ref.at[pl.ds(pl.multiple_of(b0 * ROWCAP + k0, 8), FLUSH)])
    pltpu.sync_copy(stg0.at[pl.ds(0, FLUSH)],
                    eids_ref.at[pl.ds(pl.multiple_of(b1 * ROWCAP + k1, 8), FLUSH)])
    cbuf[pl.ds(0, 16)] = jnp.full((16,), 1, jnp.int32) * k0
    pltpu.sync_copy(cbuf.at[pl.ds(0, 16)], counts_ref.at[pl.ds(b0 * 16, 16)])
    cbuf[pl.ds(0, 16)] = jnp.full((16,), 1, jnp.int32) * k1
    pltpu.sync_copy(cbuf.at[pl.ds(0, 16)], counts_ref.at[pl.ds(b1 * 16, 16)])


def _run_s0(dst_pad):
    mesh = plsc.VectorSubcoreMesh(core_axis_name="c", subcore_axis_name="s")
    f = pl.kernel(
        _s0_body,
        out_type=(
            jax.ShapeDtypeStruct((NBUCKET * ROWCAP,), jnp.int32),
            jax.ShapeDtypeStruct((NBUCKET * 16,), jnp.int32),
        ),
        mesh=mesh,
        compiler_params=pltpu.CompilerParams(needs_layout_passes=False),
        scratch_types=[
            pltpu.VMEM((CH,), jnp.int32),
            pltpu.VMEM((STG,), jnp.int32),
            pltpu.VMEM((STG,), jnp.int32),
            pltpu.VMEM((16,), jnp.int32),
        ],
    )
    return f(dst_pad)


def _s1_body(dst_ref, src_ref, c_ref, g_ref, eids_ref, counts_ref,
             sum_ref, sq_ref, mn_ref, mx_ref, cnt_ref,
             ebuf, dvals, svals, crows, grows,
             acc_s, acc_q, acc_mn, acc_mx, acc_c, cbuf, ccompact,
             sem1, sem2, sem3, sem4):
    c = lax.axis_index("c")
    s = lax.axis_index("s")
    wid = s * NC + c

    zero16 = jnp.zeros((16,), jnp.float32)
    inf16 = jnp.full((16,), jnp.inf, jnp.float32)
    ninf16 = jnp.full((16,), -jnp.inf, jnp.float32)
    one0 = (lax.iota(jnp.int32, 16) == 0).astype(jnp.float32)
    iota = lax.iota(jnp.int32, 16)

    for p in range(2):
        b = wid * 2 + p
        lo = wid * RW + p * RB

        def init_body(i, carry):
            for k in range(8):
                off = pl.ds(i * 128 + k * 16, 16)
                acc_s[off] = zero16
                acc_q[off] = zero16
                acc_mn[off] = inf16
                acc_mx[off] = ninf16
            return carry

        lax.fori_loop(0, RB + 1, init_body, 0)

        def initc(i, carry):
            acc_c[pl.ds(i * 16, 16)] = zero16
            return carry

        lax.fori_loop(0, (RB + 1), initc, 0)

        pltpu.sync_copy(counts_ref.at[pl.ds(b * 16, 16)], cbuf)
        kcount = cbuf[pl.ds(0, 16)][0]
        nb = (kcount + (CB - 1)) // CB

        def batch_body(bi, carry):
            base = bi * CB
            pltpu.sync_copy(eids_ref.at[pl.ds(b * ROWCAP + base, CB)], ebuf)
            d_dst = pltpu.async_copy(dst_ref.at[ebuf], dvals, sem1)
            d_src = pltpu.async_copy(src_ref.at[ebuf], svals, sem2)
            d_c = pltpu.async_copy(c_ref.at[ebuf], crows, sem3)
            d_src.wait()
            d_g = pltpu.async_copy(g_ref.at[svals], grows, sem4)
            d_dst.wait()
            d_c.wait()
            d_g.wait()
            limit = kcount - base

            def group_body(g, rc):
                dvec = dvals[pl.ds(g * 16, 16)]
                for kk in range(16):
                    dv = dvec[kk]
                    valid = (dv >= lo) & (dv < lo + RB) & (g * 16 + kk < limit)
                    local = jnp.where(valid, dv - lo, RB)
                    j = g * 16 + kk
                    rb = local * 128
                    for k in range(8):
                        off = pl.ds(rb + k * 16, 16)
                        ks = pl.ds(k * 16, 16)
                        mrow = jnp.maximum(crows[j, ks] + grows[j, ks], 0.0)
                        acc_s[off] = acc_s[off] + mrow
                        acc_q[off] = acc_q[off] + mrow * mrow
                        acc_mn[off] = jnp.minimum(acc_mn[off], mrow)
                        acc_mx[off] = jnp.maximum(acc_mx[off], mrow)
                    coff = pl.ds(local * 16, 16)
                    acc_c[coff] = acc_c[coff] + jnp.where(valid, one0, 0.0)
                return rc

            lax.fori_loop(0, CB // 16, group_body, 0)
            return carry

        lax.fori_loop(0, nb, batch_body, 0)

        def cgath(i, carry):
            idx = (i * 16 + iota) * 16
            ccompact[pl.ds(i * 16, 16)] = plsc.load_gather(acc_c, [idx])
            return carry

        lax.fori_loop(0, RB // 16, cgath, 0)

        pltpu.sync_copy(acc_s.at[pl.ds(0, RB * 128)],
                        sum_ref.at[pl.ds(lo * 128, RB * 128)])
        pltpu.sync_copy(acc_q.at[pl.ds(0, RB * 128)],
                        sq_ref.at[pl.ds(lo * 128, RB * 128)])
        pltpu.sync_copy(acc_mn.at[pl.ds(0, RB * 128)],
                        mn_ref.at[pl.ds(lo * 128, RB * 128)])
        pltpu.sync_copy(acc_mx.at[pl.ds(0, RB * 128)],
                        mx_ref.at[pl.ds(lo * 128, RB * 128)])
        pltpu.sync_copy(ccompact.at[pl.ds(0, RB)],
                        cnt_ref.at[pl.ds(lo, RB)])


def _run_s1(dst_pad, src_pad, c_pad, g, eids, counts):
    mesh = plsc.VectorSubcoreMesh(core_axis_name="c", subcore_axis_name="s")
    f = pl.kernel(
        _s1_body,
        out_type=(
            jax.ShapeDtypeStruct((NPAD * 128,), jnp.float32),
            jax.ShapeDtypeStruct((NPAD * 128,), jnp.float32),
            jax.ShapeDtypeStruct((NPAD * 128,), jnp.float32),
            jax.ShapeDtypeStruct((NPAD * 128,), jnp.float32),
            jax.ShapeDtypeStruct((NPAD,), jnp.float32),
        ),
        mesh=mesh,
        compiler_params=pltpu.CompilerParams(needs_layout_passes=False),
        scratch_types=[
            pltpu.VMEM((CB,), jnp.int32),
            pltpu.VMEM((CB,), jnp.int32),
            pltpu.VMEM((CB,), jnp.int32),
            pltpu.VMEM((CB, 128), jnp.float32),
            pltpu.VMEM((CB, 128), jnp.float32),
            pltpu.VMEM(((RB + 1) * 128,), jnp.float32),
            pltpu.VMEM(((RB + 1) * 128,), jnp.float32),
            pltpu.VMEM(((RB + 1) * 128,), jnp.float32),
            pltpu.VMEM(((RB + 1) * 128,), jnp.float32),
            pltpu.VMEM(((RB + 1) * 16,), jnp.float32),
            pltpu.VMEM((16,), jnp.int32),
            pltpu.VMEM((RB,), jnp.float32),
            pltpu.SemaphoreType.DMA,
            pltpu.SemaphoreType.DMA,
            pltpu.SemaphoreType.DMA,
            pltpu.SemaphoreType.DMA,
        ],
    )
    return f(dst_pad, src_pad, c_pad, g, eids, counts)


def _mm_body(x_ref, w_ref, o_ref):
    o_ref[...] = jnp.dot(x_ref[...], w_ref[...],
                         preferred_element_type=jnp.float32)


def _node_matmul(xp, w):
    din = xp.shape[1]
    return pl.pallas_call(
        _mm_body,
        grid=(NPAD // 256,),
        in_specs=[
            pl.BlockSpec((256, din), lambda i: (i, 0)),
            pl.BlockSpec((din, 128), lambda i: (0, 0)),
        ],
        out_specs=pl.BlockSpec((256, 128), lambda i: (i, 0)),
        out_shape=jax.ShapeDtypeStruct((NPAD, 128), jnp.float32),
    )(xp, w)


def _edge_body(ea_ref, w0_ref, b0_ref, w1_ref, b1_ref, c0_ref, c1_ref):
    ea = ea_ref[...]
    c0_ref[...] = jnp.dot(ea, w0_ref[...],
                          preferred_element_type=jnp.float32) + b0_ref[...]
    c1_ref[...] = jnp.dot(ea, w1_ref[...],
                          preferred_element_type=jnp.float32) + b1_ref[...]


def _edge_terms(eap, w0, b0, w1, b1):
    be = 1024
    return pl.pallas_call(
        _edge_body,
        grid=(EPAD // be,),
        in_specs=[
            pl.BlockSpec((be, ED), lambda i: (i, 0)),
            pl.BlockSpec((ED, 128), lambda i: (0, 0)),
            pl.BlockSpec((1, 128), lambda i: (0, 0)),
            pl.BlockSpec((ED, 128), lambda i: (0, 0)),
            pl.BlockSpec((1, 128), lambda i: (0, 0)),
        ],
        out_specs=[
            pl.BlockSpec((be, 128), lambda i: (i, 0)),
            pl.BlockSpec((be, 128), lambda i: (i, 0)),
        ],
        out_shape=[
            jax.ShapeDtypeStruct((EPAD, 128), jnp.float32),
            jax.ShapeDtypeStruct((EPAD, 128), jnp.float32),
        ],
    )(eap, w0, b0.reshape(1, 128), w1, b1.reshape(1, 128))


def _fin_body(with_next, sum_r, sq_r, mn_r, mx_r, cnt_r, nid_r, wp_r, bp_r,
              *rest):
    if with_next:
        wn_r, out_r = rest
    else:
        (out_r,) = rest
    s = sum_r[...]
    cnt = cnt_r[...]
    c1 = jnp.maximum(cnt, 1.0)
    mean = s / c1
    msq = sq_r[...] / c1
    std = jnp.sqrt(jnp.maximum(msq - mean * mean, 0.0) + 1e-5)
    has = cnt > 0.0
    mn = jnp.where(has, mn_r[...], 0.0)
    mx = jnp.where(has, mx_r[...], 0.0)
    agg = jnp.concatenate([s, mn, mx, std], axis=1)
    nid = nid_r[...]
    acc = jnp.zeros((256, 128), jnp.float32)
    for t in range(ND):
        msk = (nid == t).astype(jnp.float32)
        acc = acc + msk * (jnp.dot(agg, wp_r[t],
                                   preferred_element_type=jnp.float32)
                           + bp_r[t][None, :])
    if with_next:
        out_r[...] = jnp.dot(jnp.maximum(acc, 0.0), wn_r[...],
                             preferred_element_type=jnp.float32)
    else:
        out_r[...] = acc


def _finish(sums, nid_b, wp, bp, wnext=None):
    s, q, mn, mx, cnt_b = sums
    with_next = wnext is not None
    blk = lambda i: (i, 0)
    in_specs = [
        pl.BlockSpec((256, 128), blk),
        pl.BlockSpec((256, 128), blk),
        pl.BlockSpec((256, 128), blk),
        pl.BlockSpec((256, 128), blk),
        pl.BlockSpec((256, 128), blk),
        pl.BlockSpec((256, 128), blk),
        pl.BlockSpec((ND, 512, 128), lambda i: (0, 0, 0)),
        pl.BlockSpec((ND, 128), lambda i: (0, 0)),
    ]
    args = [s, q, mn, mx, cnt_b, nid_b, wp, bp]
    if with_next:
        in_specs.append(pl.BlockSpec((128, 128), lambda i: (0, 0)))
        args.append(wnext)
    return pl.pallas_call(
        functools.partial(_fin_body, with_next),
        grid=(NPAD // 256,),
        in_specs=in_specs,
        out_specs=pl.BlockSpec((256, 128), blk),
        out_shape=jax.ShapeDtypeStruct((NPAD, 128), jnp.float32),
    )(*args)


def kernel(x, edge_index, edge_attr, node_ids,
           Wm0, bm0, Wp0, bp0, Wm1, bm1, Wp1, bp1):
    xp = jnp.pad(x, ((0, NPAD - N), (0, 0)))
    eap = jnp.pad(edge_attr, ((0, EPAD - E), (0, 0)))
    dst_pad = jnp.pad(edge_index[1], (0, EPAD - E),
                      constant_values=BIGDST).astype(jnp.int32)
    src_pad = jnp.pad(edge_index[0], (0, EPAD - E)).astype(jnp.int32)
    nid_pad = jnp.pad(node_ids, (0, NPAD - N)).astype(jnp.int32)
    nid_b = jnp.broadcast_to(nid_pad[:, None], (NPAD, 128))

    g0 = _node_matmul(xp, Wm0[:D])
    c0, c1 = _edge_terms(eap, Wm0[D:], bm0, Wm1[D:], bm1)
    eids, counts = _run_s0(dst_pad)

    s0_, q0_, mn0_, mx0_, cnt0_ = _run_s1(dst_pad, src_pad, c0, g0,
                                          eids, counts)
    sums0 = (s0_.reshape(NPAD, 128), q0_.reshape(NPAD, 128),
             mn0_.reshape(NPAD, 128), mx0_.reshape(NPAD, 128),
             jnp.broadcast_to(cnt0_[:, None], (NPAD, 128)))
    g1 = _finish(sums0, nid_b, Wp0, bp0, wnext=Wm1[:D])

    s1_, q1_, mn1_, mx1_, cnt1_ = _run_s1(dst_pad, src_pad, c1, g1,
                                          eids, counts)
    sums1 = (s1_.reshape(NPAD, 128), q1_.reshape(NPAD, 128),
             mn1_.reshape(NPAD, 128), mx1_.reshape(NPAD, 128),
             jnp.broadcast_to(cnt1_[:, None], (NPAD, 128)))
    out = _finish(sums1, nid_b, Wp1, bp1, wnext=None)
    return out[:N]


# speedup vs baseline: 2.2529x; 1.4332x over previous
import functools

import jax
import jax.numpy as jnp
from jax import lax
from jax.experimental import pallas as pl
from jax.experimental.pallas import tpu as pltpu
from jax.experimental.pallas import tpu_sc as plsc

N = 10000
E = 320000
D = 128
ED = 16
ND = 20

NC = 2
NS = 16
NW = NC * NS

NPAD = 10240
RW = NPAD // NW
RB = RW // 2
NBUCKET = 2 * NW

CH = 4000
VPC = CH // 16
NCHUNK = E // CH
STG = 4048
FLUSH = 4032
ROWCAP = E + 8192
EPAD = 321536
SENT = E
CB = 64
BIGDST = 1 << 20


def _s0_body(dst_ref, src_ref, eidp_ref, srcs_ref, counts_ref,
             dbuf, sbuf, stg0, stg1, stg0s, stg1s, cbuf):
    c = lax.axis_index("c")
    s = lax.axis_index("s")
    wid = s * NC + c
    b0 = wid * 2
    b1 = wid * 2 + 1
    lo0 = wid * RW
    hi0 = lo0 + RB
    hi1 = lo0 + RW

    zero16 = jnp.zeros((16,), jnp.int32)
    sentp16 = jnp.full((16,), (RB << 19) | SENT, jnp.int32)

    def zi(i, carry):
        stg0[pl.ds(i * 16, 16)] = sentp16
        stg1[pl.ds(i * 16, 16)] = sentp16
        stg0s[pl.ds(i * 16, 16)] = zero16
        stg1s[pl.ds(i * 16, 16)] = zero16
        return carry

    lax.fori_loop(0, STG // 16, zi, 0)

    iota = lax.iota(jnp.int32, 16)

    def chunk_body(ci, carry):
        k0, k1 = carry
        pltpu.sync_copy(dst_ref.at[pl.ds(ci * CH, CH)], dbuf)
        pltpu.sync_copy(src_ref.at[pl.ds(ci * CH, CH)], sbuf)

        def vec_body(i, kk):
            kc0, kc1 = kk
            d = dbuf[pl.ds(i * 16, 16)]
            sv = sbuf[pl.ds(i * 16, 16)]
            eid = ci * CH + i * 16 + iota
            m0 = (d >= lo0) & (d < hi0)
            m1 = (d >= hi0) & (d < hi1)
            p0 = ((d - lo0) << 19) | eid
            p1 = ((d - hi0) << 19) | eid
            cs0 = plsc.cumsum(m0.astype(jnp.int32))
            cs1 = plsc.cumsum(m1.astype(jnp.int32))
            plsc.store_scatter(stg0, [kc0 + cs0 - 1], p0, mask=m0)
            plsc.store_scatter(stg0s, [kc0 + cs0 - 1], sv, mask=m0)
            plsc.store_scatter(stg1, [kc1 + cs1 - 1], p1, mask=m1)
            plsc.store_scatter(stg1s, [kc1 + cs1 - 1], sv, mask=m1)
            return kc0 + cs0[15], kc1 + cs1[15]

        kc0, kc1 = lax.fori_loop(0, VPC, vec_body,
                                 (jnp.int32(0), jnp.int32(0)))
        stg0[pl.ds(kc0, 16)] = sentp16
        stg0s[pl.ds(kc0, 16)] = zero16
        stg1[pl.ds(kc1, 16)] = sentp16
        stg1s[pl.ds(kc1, 16)] = zero16
        o0 = pl.multiple_of(b0 * ROWCAP + k0, 8)
        o1 = pl.multiple_of(b1 * ROWCAP + k1, 8)
        pltpu.sync_copy(stg0.at[pl.ds(0, FLUSH)], eidp_ref.at[pl.ds(o0, FLUSH)])
        pltpu.sync_copy(stg0s.at[pl.ds(0, FLUSH)], srcs_ref.at[pl.ds(o0, FLUSH)])
        pltpu.sync_copy(stg1.at[pl.ds(0, FLUSH)], eidp_ref.at[pl.ds(o1, FLUSH)])
        pltpu.sync_copy(stg1s.at[pl.ds(0, FLUSH)], srcs_ref.at[pl.ds(o1, FLUSH)])
        k0 = k0 + ((kc0 + 7) & (-8))
        k1 = k1 + ((kc1 + 7) & (-8))
        return k0, k1

    k0, k1 = lax.fori_loop(0, NCHUNK, chunk_body,
                           (jnp.int32(0), jnp.int32(0)))

    def sfill(i, carry):
        stg0[pl.ds(i * 16, 16)] = sentp16
        stg0s[pl.ds(i * 16, 16)] = zero16
        return carry

    lax.fori_loop(0, STG // 16, sfill, 0)
    o0 = pl.multiple_of(b0 * ROWCAP + k0, 8)
    o1 = pl.multiple_of(b1 * ROWCAP + k1, 8)
    pltpu.sync_copy(stg0.at[pl.ds(0, FLUSH)], eidp_ref.at[pl.ds(o0, FLUSH)])
    pltpu.sync_copy(stg0s.at[pl.ds(0, FLUSH)], srcs_ref.at[pl.ds(o0, FLUSH)])
    pltpu.sync_copy(stg0.at[pl.ds(0, FLUSH)], eidp_ref.at[pl.ds(o1, FLUSH)])
    pltpu.sync_copy(stg0s.at[pl.ds(0, FLUSH)], srcs_ref.at[pl.ds(o1, FLUSH)])
    cbuf[pl.ds(0, 16)] = jnp.full((16,), 1, jnp.int32) * k0
    pltpu.sync_copy(cbuf.at[pl.ds(0, 16)], counts_ref.at[pl.ds(b0 * 16, 16)])
    cbuf[pl.ds(0, 16)] = jnp.full((16,), 1, jnp.int32) * k1
    pltpu.sync_copy(cbuf.at[pl.ds(0, 16)], counts_ref.at[pl.ds(b1 * 16, 16)])


def _run_s0(dst, src):
    mesh = plsc.VectorSubcoreMesh(core_axis_name="c", subcore_axis_name="s")
    f = pl.kernel(
        _s0_body,
        out_type=(
            jax.ShapeDtypeStruct((NBUCKET * ROWCAP,), jnp.int32),
            jax.ShapeDtypeStruct((NBUCKET * ROWCAP,), jnp.int32),
            jax.ShapeDtypeStruct((NBUCKET * 16,), jnp.int32),
        ),
        mesh=mesh,
        compiler_params=pltpu.CompilerParams(needs_layout_passes=False),
        scratch_types=[
            pltpu.VMEM((CH,), jnp.int32),
            pltpu.VMEM((CH,), jnp.int32),
            pltpu.VMEM((STG,), jnp.int32),
            pltpu.VMEM((STG,), jnp.int32),
            pltpu.VMEM((STG,), jnp.int32),
            pltpu.VMEM((STG,), jnp.int32),
            pltpu.VMEM((16,), jnp.int32),
        ],
    )
    return f(dst, src)


def _s1_body(c_ref, g_ref, eidp_ref, srcs_ref, counts_ref,
             sum_ref, sq_ref, mn_ref, mx_ref, cnt_ref,
             pbuf0, ebuf0, sbuf0, crows0, grows0,
             pbuf1, ebuf1, sbuf1, crows1, grows1,
             acc_s, acc_q, acc_mn, acc_mx, acc_c, cbuf, ccompact,
             semc0, semg0, semc1, semg1):
    c = lax.axis_index("c")
    s = lax.axis_index("s")
    wid = s * NC + c

    zero16 = jnp.zeros((16,), jnp.float32)
    inf16 = jnp.full((16,), jnp.inf, jnp.float32)
    ninf16 = jnp.full((16,), -jnp.inf, jnp.float32)
    one0 = (lax.iota(jnp.int32, 16) == 0).astype(jnp.float32)
    iota = lax.iota(jnp.int32, 16)
    mask19 = jnp.full((16,), (1 << 19) - 1, jnp.int32)
    sets = ((pbuf0, ebuf0, sbuf0, crows0, grows0, semc0, semg0),
            (pbuf1, ebuf1, sbuf1, crows1, grows1, semc1, semg1))

    for p in range(2):
        b = wid * 2 + p
        lo = wid * RW + p * RB

        def init_body(i, carry):
            for k in range(8):
                off = pl.ds(i * 128 + k * 16, 16)
                acc_s[off] = zero16
                acc_q[off] = zero16
                acc_mn[off] = inf16
                acc_mx[off] = ninf16
            return carry

        lax.fori_loop(0, RB + 1, init_body, 0)

        def initc(i, carry):
            acc_c[pl.ds(i * 16, 16)] = zero16
            return carry

        lax.fori_loop(0, (RB + 1), initc, 0)

        pltpu.sync_copy(counts_ref.at[pl.ds(b * 16, 16)], cbuf)
        kcount = cbuf[pl.ds(0, 16)][0]
        nb = (kcount + (CB - 1)) // CB

        def start(bi, si):
            pbuf, ebuf, sbuf, crows, grows, semc, semg = sets[si]
            off = pl.multiple_of(b * ROWCAP + bi * CB, 8)
            pltpu.sync_copy(eidp_ref.at[pl.ds(off, CB)], pbuf)
            pltpu.sync_copy(srcs_ref.at[pl.ds(off, CB)], sbuf)
            for u in range(CB // 16):
                us = pl.ds(u * 16, 16)
                ebuf[us] = pbuf[us] & mask19
            pltpu.async_copy(c_ref.at[ebuf], crows, semc)
            pltpu.async_copy(g_ref.at[sbuf], grows, semg)

        def proc(bi, si):
            pbuf, ebuf, sbuf, crows, grows, semc, semg = sets[si]
            pltpu.make_async_copy(c_ref.at[ebuf], crows, semc).wait()
            pltpu.make_async_copy(g_ref.at[sbuf], grows, semg).wait()

            def group_body(g, rc):
                pvec = pbuf[pl.ds(g * 16, 16)]
                for kk in range(16):
                    pv = pvec[kk]
                    local = lax.shift_right_logical(pv, 19)
                    j = g * 16 + kk
                    rb = local * 128
                    for k in range(8):
                        off = pl.ds(rb + k * 16, 16)
                        ks = pl.ds(k * 16, 16)
                        mrow = jnp.maximum(crows[j, ks] + grows[j, ks], 0.0)
                        acc_s[off] = acc_s[off] + mrow
                        acc_q[off] = acc_q[off] + mrow * mrow
                        acc_mn[off] = jnp.minimum(acc_mn[off], mrow)
                        acc_mx[off] = jnp.maximum(acc_mx[off], mrow)
                    coff = pl.ds(local * 16, 16)
                    acc_c[coff] = acc_c[coff] + one0
                return rc

            lax.fori_loop(0, CB // 16, group_body, 0)

        @pl.when(nb > 0)
        def _():
            start(0, 0)

        def pair_body(h, carry):
            b0i = h * 2
            b1i = b0i + 1

            @pl.when(b1i < nb)
            def _():
                start(b1i, 1)

            proc(b0i, 0)

            @pl.when(b0i + 2 < nb)
            def _():
                start(b0i + 2, 0)

            @pl.when(b1i < nb)
            def _():
                proc(b1i, 1)

            return carry

        lax.fori_loop(0, (nb + 1) // 2, pair_body, 0)

        def cgath(i, carry):
            idx = (i * 16 + iota) * 16
            ccompact[pl.ds(i * 16, 16)] = plsc.load_gather(acc_c, [idx])
            return carry

        lax.fori_loop(0, RB // 16, cgath, 0)

        pltpu.sync_copy(acc_s.at[pl.ds(0, RB * 128)],
                        sum_ref.at[pl.ds(lo * 128, RB * 128)])
        pltpu.sync_copy(acc_q.at[pl.ds(0, RB * 128)],
                        sq_ref.at[pl.ds(lo * 128, RB * 128)])
        pltpu.sync_copy(acc_mn.at[pl.ds(0, RB * 128)],
                        mn_ref.at[pl.ds(lo * 128, RB * 128)])
        pltpu.sync_copy(acc_mx.at[pl.ds(0, RB * 128)],
                        mx_ref.at[pl.ds(lo * 128, RB * 128)])
        pltpu.sync_copy(ccompact.at[pl.ds(0, RB)],
                        cnt_ref.at[pl.ds(lo, RB)])


def _run_s1(c_pad, g, eidp, srcs, counts):
    mesh = plsc.VectorSubcoreMesh(core_axis_name="c", subcore_axis_name="s")
    f = pl.kernel(
        _s1_body,
        out_type=(
            jax.ShapeDtypeStruct((NPAD * 128,), jnp.float32),
            jax.ShapeDtypeStruct((NPAD * 128,), jnp.float32),
            jax.ShapeDtypeStruct((NPAD * 128,), jnp.float32),
            jax.ShapeDtypeStruct((NPAD * 128,), jnp.float32),
            jax.ShapeDtypeStruct((NPAD,), jnp.float32),
        ),
        mesh=mesh,
        compiler_params=pltpu.CompilerParams(needs_layout_passes=False),
        scratch_types=[
            pltpu.VMEM((CB,), jnp.int32),
            pltpu.VMEM((CB,), jnp.int32),
            pltpu.VMEM((CB,), jnp.int32),
            pltpu.VMEM((CB, 128), jnp.float32),
            pltpu.VMEM((CB, 128), jnp.float32),
            pltpu.VMEM((CB,), jnp.int32),
            pltpu.VMEM((CB,), jnp.int32),
            pltpu.VMEM((CB,), jnp.int32),
            pltpu.VMEM((CB, 128), jnp.float32),
            pltpu.VMEM((CB, 128), jnp.float32),
            pltpu.VMEM(((RB + 1) * 128,), jnp.float32),
            pltpu.VMEM(((RB + 1) * 128,), jnp.float32),
            pltpu.VMEM(((RB + 1) * 128,), jnp.float32),
            pltpu.VMEM(((RB + 1) * 128,), jnp.float32),
            pltpu.VMEM(((RB + 1) * 16,), jnp.float32),
            pltpu.VMEM((16,), jnp.int32),
            pltpu.VMEM((RB,), jnp.float32),
            pltpu.SemaphoreType.DMA,
            pltpu.SemaphoreType.DMA,
            pltpu.SemaphoreType.DMA,
            pltpu.SemaphoreType.DMA,
        ],
    )
    return f(c_pad, g, eidp, srcs, counts)


def _mm_body(x_ref, w_ref, o_ref):
    o_ref[...] = jnp.dot(x_ref[...], w_ref[...],
                         preferred_element_type=jnp.float32)


def _node_matmul(xp, w):
    din = xp.shape[1]
    return pl.pallas_call(
        _mm_body,
        grid=(NPAD // 256,),
        in_specs=[
            pl.BlockSpec((256, din), lambda i: (i, 0)),
            pl.BlockSpec((din, 128), lambda i: (0, 0)),
        ],
        out_specs=pl.BlockSpec((256, 128), lambda i: (i, 0)),
        out_shape=jax.ShapeDtypeStruct((NPAD, 128), jnp.float32),
    )(xp, w)


def _edge_body(ea_ref, w0_ref, b0_ref, w1_ref, b1_ref, c0_ref, c1_ref):
    ea = ea_ref[...]
    c0_ref[...] = jnp.dot(ea, w0_ref[...],
                          preferred_element_type=jnp.float32) + b0_ref[...]
    c1_ref[...] = jnp.dot(ea, w1_ref[...],
                          preferred_element_type=jnp.float32) + b1_ref[...]


def _edge_terms(eap, w0, b0, w1, b1):
    be = 1024
    return pl.pallas_call(
        _edge_body,
        grid=(EPAD // be,),
        in_specs=[
            pl.BlockSpec((be, ED), lambda i: (i, 0)),
            pl.BlockSpec((ED, 128), lambda i: (0, 0)),
            pl.BlockSpec((1, 128), lambda i: (0, 0)),
            pl.BlockSpec((ED, 128), lambda i: (0, 0)),
            pl.BlockSpec((1, 128), lambda i: (0, 0)),
        ],
        out_specs=[
            pl.BlockSpec((be, 128), lambda i: (i, 0)),
            pl.BlockSpec((be, 128), lambda i: (i, 0)),
        ],
        out_shape=[
            jax.ShapeDtypeStruct((EPAD, 128), jnp.float32),
            jax.ShapeDtypeStruct((EPAD, 128), jnp.float32),
        ],
    )(eap, w0, b0.reshape(1, 128), w1, b1.reshape(1, 128))


def _fin_body(with_next, sum_r, sq_r, mn_r, mx_r, cnt_r, nid_r, wp_r, bp_r,
              *rest):
    if with_next:
        wn_r, out_r = rest
    else:
        (out_r,) = rest
    s = sum_r[...]
    cnt = cnt_r[...]
    c1 = jnp.maximum(cnt, 1.0)
    mean = s / c1
    msq = sq_r[...] / c1
    std = jnp.sqrt(jnp.maximum(msq - mean * mean, 0.0) + 1e-5)
    has = cnt > 0.0
    mn = jnp.where(has, mn_r[...], 0.0)
    mx = jnp.where(has, mx_r[...], 0.0)
    agg = jnp.concatenate([s, mn, mx, std], axis=1)
    nid = nid_r[...]
    acc = jnp.zeros((256, 128), jnp.float32)
    for t in range(ND):
        msk = (nid == t).astype(jnp.float32)
        acc = acc + msk * (jnp.dot(agg, wp_r[t],
                                   preferred_element_type=jnp.float32)
                           + bp_r[t][None, :])
    if with_next:
        out_r[...] = jnp.dot(jnp.maximum(acc, 0.0), wn_r[...],
                             preferred_element_type=jnp.float32)
    else:
        out_r[...] = acc


def _finish(sums, nid_b, wp, bp, wnext=None):
    s, q, mn, mx, cnt_b = sums
    with_next = wnext is not None
    blk = lambda i: (i, 0)
    in_specs = [
        pl.BlockSpec((256, 128), blk),
        pl.BlockSpec((256, 128), blk),
        pl.BlockSpec((256, 128), blk),
        pl.BlockSpec((256, 128), blk),
        pl.BlockSpec((256, 128), blk),
        pl.BlockSpec((256, 128), blk),
        pl.BlockSpec((ND, 512, 128), lambda i: (0, 0, 0)),
        pl.BlockSpec((ND, 128), lambda i: (0, 0)),
    ]
    args = [s, q, mn, mx, cnt_b, nid_b, wp, bp]
    if with_next:
        in_specs.append(pl.BlockSpec((128, 128), lambda i: (0, 0)))
        args.append(wnext)
    return pl.pallas_call(
        functools.partial(_fin_body, with_next),
        grid=(NPAD // 256,),
        in_specs=in_specs,
        out_specs=pl.BlockSpec((256, 128), blk),
        out_shape=jax.ShapeDtypeStruct((NPAD, 128), jnp.float32),
    )(*args)


def kernel(x, edge_index, edge_attr, node_ids,
           Wm0, bm0, Wp0, bp0, Wm1, bm1, Wp1, bp1):
    xp = jnp.pad(x, ((0, NPAD - N), (0, 0)))
    eap = jnp.pad(edge_attr, ((0, EPAD - E), (0, 0)))
    dst = edge_index[1].astype(jnp.int32)
    srcv = edge_index[0].astype(jnp.int32)
    nid_pad = jnp.pad(node_ids, (0, NPAD - N)).astype(jnp.int32)
    nid_b = jnp.broadcast_to(nid_pad[:, None], (NPAD, 128))

    g0 = _node_matmul(xp, Wm0[:D])
    c0, c1 = _edge_terms(eap, Wm0[D:], bm0, Wm1[D:], bm1)
    eidp, srcs, counts = _run_s0(dst, srcv)

    s0_, q0_, mn0_, mx0_, cnt0_ = _run_s1(c0, g0, eidp, srcs, counts)
    sums0 = (s0_.reshape(NPAD, 128), q0_.reshape(NPAD, 128),
             mn0_.reshape(NPAD, 128), mx0_.reshape(NPAD, 128),
             jnp.broadcast_to(cnt0_[:, None], (NPAD, 128)))
    g1 = _finish(sums0, nid_b, Wp0, bp0, wnext=Wm1[:D])

    s1_, q1_, mn1_, mx1_, cnt1_ = _run_s1(c1, g1, eidp, srcs, counts)
    sums1 = (s1_.reshape(NPAD, 128), q1_.reshape(NPAD, 128),
             mn1_.reshape(NPAD, 128), mx1_.reshape(NPAD, 128),
             jnp.broadcast_to(cnt1_[:, None], (NPAD, 128)))
    out = _finish(sums1, nid_b, Wp1, bp1, wnext=None)
    return out[:N]
